# baseline (speedup 1.0000x reference)
import functools

import jax
import jax.numpy as jnp
from jax import lax
from jax.experimental import pallas as pl
from jax.experimental.pallas import tpu as pltpu
from jax.experimental.pallas import tpu_sc as plsc

_PB = 1280


def _prefix_body(x_ref, t_ref, out_ref, carry_ref):
    @pl.when(pl.program_id(0) == 0)
    def _():
        carry_ref[...] = jnp.zeros_like(carry_ref)

    x = x_ref[...] - 0.5
    nrow = x.shape[0]
    exc = jax.lax.dot(
        t_ref[...], x,
        precision=jax.lax.Precision.HIGHEST,
        preferred_element_type=jnp.float32,
    )
    out_ref[...] = carry_ref[0:1, :] + exc
    carry_ref[0:1, :] = (
        carry_ref[0:1, :] + exc[nrow - 1 : nrow, :] + x[nrow - 1 : nrow, :]
    )


def _prefix_sum_centered(x, tril):
    n, d = x.shape
    nb = n // _PB
    return pl.pallas_call(
        _prefix_body,
        grid=(nb,),
        in_specs=[
            pl.BlockSpec((_PB, d), lambda i: (i, 0)),
            pl.BlockSpec((_PB, _PB), lambda i: (0, 0)),
        ],
        out_specs=pl.BlockSpec((_PB, d), lambda i: (i, 0)),
        out_shape=jax.ShapeDtypeStruct((n, d), jnp.float32),
        scratch_shapes=[pltpu.VMEM((8, d), jnp.float32)],
        compiler_params=pltpu.CompilerParams(dimension_semantics=("arbitrary",)),
    )(x, tril)


_SEG_CHUNK = 400


def _make_sc_gather_diff(n_rows, d, n_seg):
    info = plsc.get_sparse_core_info()
    n_cores, n_sub = info.num_cores, info.num_subcores
    nw = n_cores * n_sub
    c_sz = _SEG_CHUNK
    gp = c_sz + 8
    nchunk = n_seg // c_sz
    mesh = plsc.VectorSubcoreMesh(core_axis_name="c", subcore_axis_name="s")

    @functools.partial(
        pl.kernel,
        mesh=mesh,
        out_type=jax.ShapeDtypeStruct((n_seg, d), jnp.float32),
        scratch_types=[
            pltpu.VMEM((gp,), jnp.int32),
            pltpu.VMEM((gp, d), jnp.float32),
            pltpu.VMEM((c_sz, d), jnp.float32),
            pltpu.SemaphoreType.DMA,
        ],
    )
    def k(p_hbm, idx_hbm, out_hbm, idx_v, g_v, d_v, sem):
        wid = lax.axis_index("s") * n_cores + lax.axis_index("c")
        rem = nchunk % nw
        nloc = jnp.where(wid < rem, nchunk // nw + 1, nchunk // nw)

        def chunk_body(j, carry):
            chunk = wid + j * nw
            base = chunk * c_sz
            pltpu.sync_copy(idx_hbm.at[pl.ds(base, gp)], idx_v)
            pltpu.async_copy(p_hbm.at[idx_v], g_v, sem).wait()

            def row_body(r, c2):
                for g8 in range(d // 16):
                    off = g8 * 16
                    d_v[r, pl.ds(off, 16)] = (
                        g_v[r + 1, pl.ds(off, 16)] - g_v[r, pl.ds(off, 16)]
                    )
                return c2

            lax.fori_loop(0, c_sz, row_body, 0)
            pltpu.sync_copy(d_v, out_hbm.at[pl.ds(base, c_sz)])
            return carry

        lax.fori_loop(0, nloc, chunk_body, 0)

    return k


_VT = 200


def _norm_body(s_ref, len_ref, lib_ref, out_ref):
    s = s_ref[...] + 0.5 * len_ref[...]
    ve = jnp.log1p(s / lib_ref[...][:, None, :]) - 2.0
    n_c = ve.shape[0]
    mu = jnp.mean(ve, axis=0, keepdims=True)
    sd = jnp.sqrt(jnp.sum((ve - mu) ** 2, axis=0, keepdims=True) / (n_c - 1))
    rel = (ve - mu) / (sd + 1e-5)
    d = ve.shape[-1]
    out_ref[..., 0:d] = ve
    out_ref[..., d : 2 * d] = rel


def _normalize(seg_sums3, lens3, lib2d):
    n_c, n_v, d = seg_sums3.shape
    nt = n_v // _VT
    return pl.pallas_call(
        _norm_body,
        grid=(nt,),
        in_specs=[
            pl.BlockSpec((n_c, _VT, d), lambda t: (0, t, 0)),
            pl.BlockSpec((n_c, _VT, 1), lambda t: (0, t, 0)),
            pl.BlockSpec((n_c, d), lambda t: (0, 0)),
        ],
        out_specs=pl.BlockSpec((n_c, _VT, 2 * d), lambda t: (0, t, 0)),
        out_shape=jax.ShapeDtypeStruct((n_c, n_v, 2 * d), jnp.float32),
        compiler_params=pltpu.CompilerParams(dimension_semantics=("parallel",)),
    )(seg_sums3, lens3, lib2d)


def kernel(cut_embedding, local_clusterxvariant_indptr, n_variants, n_clusters, cluster_cut_lib):
    n_rows, d = cut_embedding.shape
    n_c = cluster_cut_lib.shape[0]
    n_seg = local_clusterxvariant_indptr.shape[0] - 1
    n_v = n_seg // n_c

    idx = local_clusterxvariant_indptr.astype(jnp.int32)
    pad = 8 - (idx.shape[0] % 8) if idx.shape[0] % 8 else 0
    idx_pad = jnp.concatenate([idx, jnp.broadcast_to(idx[-1:], (pad,))])
    lens3 = (idx[1:] - idx[:-1]).astype(jnp.float32).reshape(n_c, n_v, 1)
    lib2d = jnp.broadcast_to(
        cluster_cut_lib.astype(jnp.float32)[:, None], (n_c, d)
    )

    tril = jnp.tril(jnp.ones((_PB, _PB), jnp.float32), k=-1)
    p_centered = _prefix_sum_centered(cut_embedding.astype(jnp.float32), tril)
    seg_sums = _make_sc_gather_diff(n_rows, d, n_seg)(p_centered, idx_pad)
    return _normalize(seg_sums.reshape(n_c, n_v, d), lens3, lib2d)


# speedup vs baseline: 3.3127x; 3.3127x over previous
import functools

import jax
import jax.numpy as jnp
from jax import lax
from jax.experimental import pallas as pl
from jax.experimental.pallas import tpu as pltpu
from jax.experimental.pallas import tpu_sc as plsc

_PB = 1280


def _prefix_body(x_ref, t_ref, out_ref, carry_ref):
    @pl.when(pl.program_id(0) == 0)
    def _():
        carry_ref[...] = jnp.zeros_like(carry_ref)

    x = x_ref[...] - 0.5
    nrow = x.shape[0]
    exc = jax.lax.dot(
        t_ref[...], x.astype(jnp.bfloat16),
        preferred_element_type=jnp.float32,
    )
    out_ref[...] = carry_ref[0:1, :] + exc
    carry_ref[0:1, :] = (
        carry_ref[0:1, :] + exc[nrow - 1 : nrow, :] + x[nrow - 1 : nrow, :]
    )


def _prefix_sum_centered(x, tril):
    n, d = x.shape
    nb = n // _PB
    return pl.pallas_call(
        _prefix_body,
        grid=(nb,),
        in_specs=[
            pl.BlockSpec((_PB, d), lambda i: (i, 0)),
            pl.BlockSpec((_PB, _PB), lambda i: (0, 0)),
        ],
        out_specs=pl.BlockSpec((_PB, d), lambda i: (i, 0)),
        out_shape=jax.ShapeDtypeStruct((n, d), jnp.float32),
        scratch_shapes=[pltpu.VMEM((8, d), jnp.float32)],
        compiler_params=pltpu.CompilerParams(dimension_semantics=("arbitrary",)),
    )(x, tril)


_SEG_CHUNK = 400


def _make_sc_gather_diff(n_rows, d, n_seg):
    info = plsc.get_sparse_core_info()
    n_cores, n_sub = info.num_cores, info.num_subcores
    nw = n_cores * n_sub
    c_sz = _SEG_CHUNK
    gp = c_sz + 8
    nchunk = n_seg // c_sz
    mesh = plsc.VectorSubcoreMesh(core_axis_name="c", subcore_axis_name="s")

    @functools.partial(
        pl.kernel,
        mesh=mesh,
        out_type=jax.ShapeDtypeStruct((n_seg, d), jnp.float32),
        scratch_types=[
            pltpu.VMEM((gp,), jnp.int32),
            pltpu.VMEM((gp, d), jnp.float32),
            pltpu.VMEM((c_sz, d), jnp.float32),
            pltpu.SemaphoreType.DMA,
        ],
    )
    def k(p_hbm, idx_hbm, out_hbm, idx_v, g_v, d_v, sem):
        wid = lax.axis_index("s") * n_cores + lax.axis_index("c")
        rem = nchunk % nw
        nloc = jnp.where(wid < rem, nchunk // nw + 1, nchunk // nw)

        def chunk_body(j, carry):
            chunk = wid + j * nw
            base = chunk * c_sz
            pltpu.sync_copy(idx_hbm.at[pl.ds(base, gp)], idx_v)
            pltpu.async_copy(p_hbm.at[idx_v], g_v, sem).wait()

            def row_body(r, c2):
                for g8 in range(d // 16):
                    off = g8 * 16
                    d_v[r, pl.ds(off, 16)] = (
                        g_v[r + 1, pl.ds(off, 16)] - g_v[r, pl.ds(off, 16)]
                    )
                return c2

            lax.fori_loop(0, c_sz, row_body, 0)
            pltpu.sync_copy(d_v, out_hbm.at[pl.ds(base, c_sz)])
            return carry

        lax.fori_loop(0, nloc, chunk_body, 0)

    return k


_VT = 200


def _norm_body(s_ref, len_ref, lib_ref, out_ref):
    s = s_ref[...] + 0.5 * len_ref[...]
    ve = jnp.log1p(s / lib_ref[...][:, None, :]) - 2.0
    n_c = ve.shape[0]
    mu = jnp.mean(ve, axis=0, keepdims=True)
    sd = jnp.sqrt(jnp.sum((ve - mu) ** 2, axis=0, keepdims=True) / (n_c - 1))
    rel = (ve - mu) / (sd + 1e-5)
    d = ve.shape[-1]
    out_ref[..., 0:d] = ve
    out_ref[..., d : 2 * d] = rel


def _normalize(seg_sums3, lens3, lib2d):
    n_c, n_v, d = seg_sums3.shape
    nt = n_v // _VT
    return pl.pallas_call(
        _norm_body,
        grid=(nt,),
        in_specs=[
            pl.BlockSpec((n_c, _VT, d), lambda t: (0, t, 0)),
            pl.BlockSpec((n_c, _VT, 1), lambda t: (0, t, 0)),
            pl.BlockSpec((n_c, d), lambda t: (0, 0)),
        ],
        out_specs=pl.BlockSpec((n_c, _VT, 2 * d), lambda t: (0, t, 0)),
        out_shape=jax.ShapeDtypeStruct((n_c, n_v, 2 * d), jnp.float32),
        compiler_params=pltpu.CompilerParams(dimension_semantics=("parallel",)),
    )(seg_sums3, lens3, lib2d)


def kernel(cut_embedding, local_clusterxvariant_indptr, n_variants, n_clusters, cluster_cut_lib):
    n_rows, d = cut_embedding.shape
    n_c = cluster_cut_lib.shape[0]
    n_seg = local_clusterxvariant_indptr.shape[0] - 1
    n_v = n_seg // n_c

    idx = local_clusterxvariant_indptr.astype(jnp.int32)
    pad = 8 - (idx.shape[0] % 8) if idx.shape[0] % 8 else 0
    idx_pad = jnp.concatenate([idx, jnp.broadcast_to(idx[-1:], (pad,))])
    lens3 = (idx[1:] - idx[:-1]).astype(jnp.float32).reshape(n_c, n_v, 1)
    lib2d = jnp.broadcast_to(
        cluster_cut_lib.astype(jnp.float32)[:, None], (n_c, d)
    )

    tril = jnp.tril(jnp.ones((_PB, _PB), jnp.bfloat16), k=-1)
    p_centered = _prefix_sum_centered(cut_embedding.astype(jnp.float32), tril)
    seg_sums = _make_sc_gather_diff(n_rows, d, n_seg)(p_centered, idx_pad)
    return _normalize(seg_sums.reshape(n_c, n_v, d), lens3, lib2d)


# speedup vs baseline: 4.5034x; 1.3594x over previous
import functools

import jax
import jax.numpy as jnp
from jax import lax
from jax.experimental import pallas as pl
from jax.experimental.pallas import tpu as pltpu
from jax.experimental.pallas import tpu_sc as plsc

_SEG_CHUNK = 400
_IDXW = 416
_WIN = 128
_ACC = 408


def _make_sc_segment_sum(n_rows, d, n_seg):
    info = plsc.get_sparse_core_info()
    n_cores, n_sub = info.num_cores, info.num_subcores
    nw = n_cores * n_sub
    nchunk = n_seg // _SEG_CHUNK
    mesh = plsc.VectorSubcoreMesh(core_axis_name="c", subcore_axis_name="s")

    @functools.partial(
        pl.kernel,
        mesh=mesh,
        compiler_params=pltpu.CompilerParams(needs_layout_passes=False),
        out_type=jax.ShapeDtypeStruct((n_seg, d), jnp.float32),
        scratch_types=[
            pltpu.VMEM((_IDXW,), jnp.int32),
            pltpu.VMEM((_WIN, d), jnp.float32),
            pltpu.VMEM((_WIN,), jnp.int32),
            pltpu.VMEM((_ACC, d), jnp.float32),
            pltpu.VMEM_SHARED((16 * _ACC, d), jnp.float32),
        ],
    )
    def k(emb_hbm, idx_hbm, out_hbm, idx_v, rows_v, ids_v, zeros_v, acc):
        wid = lax.axis_index("s") * n_cores + lax.axis_index("c")
        sid = lax.axis_index("s")
        slot0 = sid * _ACC
        rem = nchunk % nw
        nloc = jnp.where(wid < rem, nchunk // nw + 1, nchunk // nw)

        lane = lax.iota(jnp.int32, 16)
        zrow = jnp.zeros((16,), jnp.float32)

        def lane0_scalar(vec16):
            return vec16[0]

        def zbody(r, c0):
            for g in range(d // 16):
                zeros_v[r, pl.ds(g * 16, 16)] = zrow
            return c0

        lax.fori_loop(0, _ACC, zbody, 0)

        def chunk_body(j, carry):
            chunk = wid + j * nw
            base = chunk * _SEG_CHUNK
            pltpu.sync_copy(idx_hbm.at[pl.ds(base, _IDXW)], idx_v)
            pltpu.sync_copy(zeros_v, acc.at[pl.ds(slot0, _ACC)])
            r0 = lane0_scalar(idx_v[pl.ds(0, 16)])
            r1 = lane0_scalar(idx_v[pl.ds(_SEG_CHUNK, 16)])
            wlo = r0 // _WIN
            whi = (r1 + (_WIN - 1)) // _WIN

            def win_body(w, c1):
                pltpu.sync_copy(emb_hbm.at[pl.ds(w * _WIN, _WIN)], rows_v)
                wbase = w * _WIN
                for g in range(_WIN // 16):
                    rowvec = wbase + g * 16 + lane
                    c = jnp.zeros((16,), jnp.int32)
                    step = 256
                    while step >= 1:
                        probe = c + (step - 1)
                        probe_c = jnp.minimum(probe, _IDXW - 1)
                        vals = plsc.load_gather(idx_v, [probe_c])
                        ok = (vals <= rowvec) & (probe <= _IDXW - 1)
                        c = jnp.where(ok, probe + 1, c)
                        step //= 2
                    f = c - 1
                    valid = (f >= 0) & (f < _SEG_CHUNK)
                    ids_v[pl.ds(g * 16, 16)] = jnp.where(
                        valid, f, _SEG_CHUNK + 4
                    ) + slot0
                pltpu.sync_copy(rows_v, acc.at[ids_v], add=True)
                return c1

            lax.fori_loop(wlo, whi, win_body, 0)
            pltpu.sync_copy(
                acc.at[pl.ds(slot0, _SEG_CHUNK)],
                out_hbm.at[pl.ds(base, _SEG_CHUNK)],
            )
            return carry

        lax.fori_loop(0, nloc, chunk_body, 0)

    return k


_VT = 200


def _norm_body(s_ref, lib_ref, out_ref):
    s = s_ref[...]
    ve = jnp.log1p(s / lib_ref[...][:, None, :]) - 2.0
    n_c = ve.shape[0]
    mu = jnp.mean(ve, axis=0, keepdims=True)
    sd = jnp.sqrt(jnp.sum((ve - mu) ** 2, axis=0, keepdims=True) / (n_c - 1))
    rel = (ve - mu) / (sd + 1e-5)
    d = ve.shape[-1]
    out_ref[..., 0:d] = ve
    out_ref[..., d : 2 * d] = rel


def _normalize(seg_sums3, lib2d):
    n_c, n_v, d = seg_sums3.shape
    nt = n_v // _VT
    return pl.pallas_call(
        _norm_body,
        grid=(nt,),
        in_specs=[
            pl.BlockSpec((n_c, _VT, d), lambda t: (0, t, 0)),
            pl.BlockSpec((n_c, d), lambda t: (0, 0)),
        ],
        out_specs=pl.BlockSpec((n_c, _VT, 2 * d), lambda t: (0, t, 0)),
        out_shape=jax.ShapeDtypeStruct((n_c, n_v, 2 * d), jnp.float32),
        compiler_params=pltpu.CompilerParams(dimension_semantics=("parallel",)),
    )(seg_sums3, lib2d)


def kernel(cut_embedding, local_clusterxvariant_indptr, n_variants, n_clusters, cluster_cut_lib):
    n_rows, d = cut_embedding.shape
    n_c = cluster_cut_lib.shape[0]
    n_seg = local_clusterxvariant_indptr.shape[0] - 1
    n_v = n_seg // n_c

    idx = local_clusterxvariant_indptr.astype(jnp.int32)
    npad = (n_seg - _SEG_CHUNK) + _IDXW - idx.shape[0]
    idx_pad = jnp.concatenate(
        [idx, jnp.broadcast_to(idx[-1:], (npad,))]
    )
    lib2d = jnp.broadcast_to(
        cluster_cut_lib.astype(jnp.float32)[:, None], (n_c, d)
    )

    seg_sums = _make_sc_segment_sum(n_rows, d, n_seg)(
        cut_embedding.astype(jnp.float32), idx_pad
    )
    return _normalize(seg_sums.reshape(n_c, n_v, d), lib2d)


# speedup vs baseline: 7.0266x; 1.5603x over previous
import functools

import jax
import jax.numpy as jnp
from jax import lax
from jax.experimental import pallas as pl
from jax.experimental.pallas import tpu as pltpu
from jax.experimental.pallas import tpu_sc as plsc

_SEG_CHUNK = 400
_IDXW = 416
_WIN = 128
_ACC = 408


def _make_sc_segment_sum(n_rows, d, n_seg):
    info = plsc.get_sparse_core_info()
    n_cores, n_sub = info.num_cores, info.num_subcores
    nw = n_cores * n_sub
    nchunk = n_seg // _SEG_CHUNK
    mesh = plsc.VectorSubcoreMesh(core_axis_name="c", subcore_axis_name="s")

    @functools.partial(
        pl.kernel,
        mesh=mesh,
        compiler_params=pltpu.CompilerParams(needs_layout_passes=False),
        out_type=jax.ShapeDtypeStruct((n_seg, d), jnp.float32),
        scratch_types=[
            pltpu.VMEM((_IDXW,), jnp.int32),
            pltpu.VMEM((_WIN, d), jnp.float32),
            pltpu.VMEM((_WIN, d), jnp.float32),
            pltpu.VMEM((_WIN,), jnp.int32),
            pltpu.VMEM((_ACC // 3, d), jnp.float32),
            pltpu.VMEM_SHARED((16 * _ACC, d), jnp.float32),
            pltpu.SemaphoreType.DMA,
            pltpu.SemaphoreType.DMA,
        ],
    )
    def k(emb_hbm, idx_hbm, out_hbm, idx_v, rows_a, rows_b, ids_v, zeros_v,
          acc, sem_a, sem_b):
        wid = lax.axis_index("s") * n_cores + lax.axis_index("c")
        sid = lax.axis_index("s")
        slot0 = sid * _ACC
        rem = nchunk % nw
        nloc = jnp.where(wid < rem, nchunk // nw + 1, nchunk // nw)

        lane = lax.iota(jnp.int32, 16)
        zrow = jnp.zeros((16,), jnp.float32)

        def lane0_scalar(vec16):
            return vec16[0]

        def zbody(r, c0):
            for g in range(d // 16):
                zeros_v[r, pl.ds(g * 16, 16)] = zrow
            return c0

        lax.fori_loop(0, _ACC // 3, zbody, 0)

        def compute_ids(w):
            wbase = w * _WIN
            for g in range(_WIN // 16):
                rowvec = wbase + g * 16 + lane
                c = jnp.zeros((16,), jnp.int32)
                step = 512
                while step >= 1:
                    probe = c + (step - 1)
                    probe_c = jnp.minimum(probe, _IDXW - 1)
                    vals = plsc.load_gather(idx_v, [probe_c])
                    ok = (vals <= rowvec) & (probe <= _IDXW - 1)
                    c = jnp.where(ok, probe + 1, c)
                    step //= 2
                f = c - 1
                valid = (f >= 0) & (f < _SEG_CHUNK)
                ids_v[pl.ds(g * 16, 16)] = jnp.where(
                    valid, f, _SEG_CHUNK + 4
                ) + slot0

        def issue(w, buf, sem):
            pltpu.async_copy(emb_hbm.at[pl.ds(w * _WIN, _WIN)], buf, sem)

        def drain(w, buf, sem):
            pltpu.make_async_copy(
                emb_hbm.at[pl.ds(w * _WIN, _WIN)], buf, sem
            ).wait()
            pltpu.sync_copy(buf, acc.at[ids_v], add=True)

        def chunk_body(j, carry):
            chunk = wid + j * nw
            base = chunk * _SEG_CHUNK
            pltpu.sync_copy(idx_hbm.at[pl.ds(base, _IDXW)], idx_v)
            r0 = lane0_scalar(idx_v[pl.ds(0, 16)])
            r1 = lane0_scalar(idx_v[pl.ds(_SEG_CHUNK, 16)])
            wlo = r0 // _WIN
            whi = (r1 + (_WIN - 1)) // _WIN

            @pl.when(whi > wlo)
            def _():
                issue(wlo, rows_a, sem_a)

            for z in range(3):
                pltpu.sync_copy(
                    zeros_v, acc.at[pl.ds(slot0 + z * (_ACC // 3), _ACC // 3)]
                )

            def pair_body(p, c1):
                w0 = wlo + 2 * p

                @pl.when(w0 + 1 < whi)
                def _():
                    issue(w0 + 1, rows_b, sem_b)

                compute_ids(w0)
                drain(w0, rows_a, sem_a)

                @pl.when(w0 + 2 < whi)
                def _():
                    issue(w0 + 2, rows_a, sem_a)

                @pl.when(w0 + 1 < whi)
                def _():
                    compute_ids(w0 + 1)
                    drain(w0 + 1, rows_b, sem_b)

                return c1

            lax.fori_loop(0, (whi - wlo + 1) // 2, pair_body, 0)
            pltpu.sync_copy(
                acc.at[pl.ds(slot0, _SEG_CHUNK)],
                out_hbm.at[pl.ds(base, _SEG_CHUNK)],
            )
            return carry

        lax.fori_loop(0, nloc, chunk_body, 0)

    return k


_VT = 200


def _norm_body(s_ref, lib_ref, out_ref):
    s = s_ref[...]
    ve = jnp.log1p(s / lib_ref[...][:, None, :]) - 2.0
    n_c = ve.shape[0]
    mu = jnp.mean(ve, axis=0, keepdims=True)
    sd = jnp.sqrt(jnp.sum((ve - mu) ** 2, axis=0, keepdims=True) / (n_c - 1))
    rel = (ve - mu) / (sd + 1e-5)
    d = ve.shape[-1]
    out_ref[..., 0:d] = ve
    out_ref[..., d : 2 * d] = rel


def _normalize(seg_sums3, lib2d):
    n_c, n_v, d = seg_sums3.shape
    nt = n_v // _VT
    return pl.pallas_call(
        _norm_body,
        grid=(nt,),
        in_specs=[
            pl.BlockSpec((n_c, _VT, d), lambda t: (0, t, 0)),
            pl.BlockSpec((n_c, d), lambda t: (0, 0)),
        ],
        out_specs=pl.BlockSpec((n_c, _VT, 2 * d), lambda t: (0, t, 0)),
        out_shape=jax.ShapeDtypeStruct((n_c, n_v, 2 * d), jnp.float32),
        compiler_params=pltpu.CompilerParams(dimension_semantics=("parallel",)),
    )(seg_sums3, lib2d)


def kernel(cut_embedding, local_clusterxvariant_indptr, n_variants, n_clusters, cluster_cut_lib):
    n_rows, d = cut_embedding.shape
    n_c = cluster_cut_lib.shape[0]
    n_seg = local_clusterxvariant_indptr.shape[0] - 1
    n_v = n_seg // n_c

    idx = local_clusterxvariant_indptr.astype(jnp.int32)
    npad = (n_seg - _SEG_CHUNK) + _IDXW - idx.shape[0]
    idx_pad = jnp.concatenate(
        [idx, jnp.broadcast_to(idx[-1:], (npad,))]
    )
    lib2d = jnp.broadcast_to(
        cluster_cut_lib.astype(jnp.float32)[:, None], (n_c, d)
    )

    seg_sums = _make_sc_segment_sum(n_rows, d, n_seg)(
        cut_embedding.astype(jnp.float32), idx_pad
    )
    return _normalize(seg_sums.reshape(n_c, n_v, d), lib2d)


# speedup vs baseline: 7.0877x; 1.0087x over previous
import functools

import jax
import jax.numpy as jnp
from jax import lax
from jax.experimental import pallas as pl
from jax.experimental.pallas import tpu as pltpu
from jax.experimental.pallas import tpu_sc as plsc

_SEG_CHUNK = 400
_IDXW = 416
_WIN = 128
_ACC = 408


def _make_sc_segment_sum(n_rows, d, n_seg):
    info = plsc.get_sparse_core_info()
    n_cores, n_sub = info.num_cores, info.num_subcores
    nw = n_cores * n_sub
    nchunk = n_seg // _SEG_CHUNK
    mesh = plsc.VectorSubcoreMesh(core_axis_name="c", subcore_axis_name="s")

    @functools.partial(
        pl.kernel,
        mesh=mesh,
        compiler_params=pltpu.CompilerParams(needs_layout_passes=False),
        out_type=jax.ShapeDtypeStruct((n_seg, d), jnp.float32),
        scratch_types=[
            pltpu.VMEM((_IDXW,), jnp.int32),
            pltpu.VMEM((_WIN, d), jnp.float32),
            pltpu.VMEM((_WIN, d), jnp.float32),
            pltpu.VMEM((_WIN, d), jnp.float32),
            pltpu.VMEM((_WIN,), jnp.int32),
            pltpu.VMEM((_ACC // 3, d), jnp.float32),
            pltpu.VMEM_SHARED((16 * _ACC, d), jnp.float32),
            pltpu.SemaphoreType.DMA,
            pltpu.SemaphoreType.DMA,
            pltpu.SemaphoreType.DMA,
            pltpu.SemaphoreType.DMA,
        ],
    )
    def k(emb_hbm, idx_hbm, out_hbm, idx_v, rows_a, rows_b, rows_c, ids_v,
          zeros_v, acc, sem_a, sem_b, sem_c, sem_o):
        wid = lax.axis_index("s") * n_cores + lax.axis_index("c")
        sid = lax.axis_index("s")
        slot0 = sid * _ACC
        rem = nchunk % nw
        nloc = jnp.where(wid < rem, nchunk // nw + 1, nchunk // nw)

        lane = lax.iota(jnp.int32, 16)
        zrow = jnp.zeros((16,), jnp.float32)

        def lane0_scalar(vec16):
            return vec16[0]

        def zbody(r, c0):
            for g in range(d // 16):
                zeros_v[r, pl.ds(g * 16, 16)] = zrow
            return c0

        lax.fori_loop(0, _ACC // 3, zbody, 0)

        def compute_ids(w):
            wbase = w * _WIN
            for g in range(_WIN // 16):
                rowvec = wbase + g * 16 + lane
                c = jnp.zeros((16,), jnp.int32)
                step = 512
                while step >= 1:
                    probe = c + (step - 1)
                    probe_c = jnp.minimum(probe, _IDXW - 1)
                    vals = plsc.load_gather(idx_v, [probe_c])
                    ok = (vals <= rowvec) & (probe <= _IDXW - 1)
                    c = jnp.where(ok, probe + 1, c)
                    step //= 2
                f = c - 1
                valid = (f >= 0) & (f < _SEG_CHUNK)
                ids_v[pl.ds(g * 16, 16)] = jnp.where(
                    valid, f, _SEG_CHUNK + 4
                ) + slot0

        def issue(w, buf, sem):
            pltpu.async_copy(emb_hbm.at[pl.ds(w * _WIN, _WIN)], buf, sem)

        def drain(w, buf, sem):
            pltpu.make_async_copy(
                emb_hbm.at[pl.ds(w * _WIN, _WIN)], buf, sem
            ).wait()
            pltpu.sync_copy(buf, acc.at[ids_v], add=True)

        def chunk_body(j, carry):
            chunk = wid + j * nw
            base = chunk * _SEG_CHUNK
            pltpu.sync_copy(idx_hbm.at[pl.ds(base, _IDXW)], idx_v)
            r0 = lane0_scalar(idx_v[pl.ds(0, 16)])
            r1 = lane0_scalar(idx_v[pl.ds(_SEG_CHUNK, 16)])
            wlo = r0 // _WIN
            whi = (r1 + (_WIN - 1)) // _WIN
            nwin = whi - wlo

            @pl.when(nwin >= 1)
            def _():
                issue(wlo, rows_a, sem_a)

            @pl.when(nwin >= 2)
            def _():
                issue(wlo + 1, rows_b, sem_b)

            @pl.when(j > 0)
            def _():
                pltpu.make_async_copy(
                    acc.at[pl.ds(slot0, _SEG_CHUNK)],
                    out_hbm.at[pl.ds((chunk - nw) * _SEG_CHUNK, _SEG_CHUNK)],
                    sem_o,
                ).wait()

            for z in range(3):
                pltpu.sync_copy(
                    zeros_v, acc.at[pl.ds(slot0 + z * (_ACC // 3), _ACC // 3)]
                )

            def tri_body(p, c1):
                w = wlo + 3 * p

                @pl.when(w + 2 < whi)
                def _():
                    issue(w + 2, rows_c, sem_c)

                compute_ids(w)
                drain(w, rows_a, sem_a)

                @pl.when(w + 3 < whi)
                def _():
                    issue(w + 3, rows_a, sem_a)

                @pl.when(w + 1 < whi)
                def _():
                    compute_ids(w + 1)
                    drain(w + 1, rows_b, sem_b)

                @pl.when(w + 4 < whi)
                def _():
                    issue(w + 4, rows_b, sem_b)

                @pl.when(w + 2 < whi)
                def _():
                    compute_ids(w + 2)
                    drain(w + 2, rows_c, sem_c)

                return c1

            lax.fori_loop(0, (nwin + 2) // 3, tri_body, 0)
            pltpu.async_copy(
                acc.at[pl.ds(slot0, _SEG_CHUNK)],
                out_hbm.at[pl.ds(base, _SEG_CHUNK)],
                sem_o,
            )
            return carry

        lax.fori_loop(0, nloc, chunk_body, 0)

        @pl.when(nloc > 0)
        def _():
            pltpu.make_async_copy(
                acc.at[pl.ds(slot0, _SEG_CHUNK)],
                out_hbm.at[
                    pl.ds((wid + (nloc - 1) * nw) * _SEG_CHUNK, _SEG_CHUNK)
                ],
                sem_o,
            ).wait()

    return k


_VT = 200


def _norm_body(s_ref, lib_ref, out_ref):
    s = s_ref[...]
    ve = jnp.log1p(s / lib_ref[...][:, None, :]) - 2.0
    n_c = ve.shape[0]
    mu = jnp.mean(ve, axis=0, keepdims=True)
    sd = jnp.sqrt(jnp.sum((ve - mu) ** 2, axis=0, keepdims=True) / (n_c - 1))
    rel = (ve - mu) / (sd + 1e-5)
    d = ve.shape[-1]
    out_ref[..., 0:d] = ve
    out_ref[..., d : 2 * d] = rel


def _normalize(seg_sums3, lib2d):
    n_c, n_v, d = seg_sums3.shape
    nt = n_v // _VT
    return pl.pallas_call(
        _norm_body,
        grid=(nt,),
        in_specs=[
            pl.BlockSpec((n_c, _VT, d), lambda t: (0, t, 0)),
            pl.BlockSpec((n_c, d), lambda t: (0, 0)),
        ],
        out_specs=pl.BlockSpec((n_c, _VT, 2 * d), lambda t: (0, t, 0)),
        out_shape=jax.ShapeDtypeStruct((n_c, n_v, 2 * d), jnp.float32),
        compiler_params=pltpu.CompilerParams(dimension_semantics=("parallel",)),
    )(seg_sums3, lib2d)


def kernel(cut_embedding, local_clusterxvariant_indptr, n_variants, n_clusters, cluster_cut_lib):
    n_rows, d = cut_embedding.shape
    n_c = cluster_cut_lib.shape[0]
    n_seg = local_clusterxvariant_indptr.shape[0] - 1
    n_v = n_seg // n_c

    idx = local_clusterxvariant_indptr.astype(jnp.int32)
    npad = (n_seg - _SEG_CHUNK) + _IDXW - idx.shape[0]
    idx_pad = jnp.concatenate(
        [idx, jnp.broadcast_to(idx[-1:], (npad,))]
    )
    lib2d = jnp.broadcast_to(
        cluster_cut_lib.astype(jnp.float32)[:, None], (n_c, d)
    )

    seg_sums = _make_sc_segment_sum(n_rows, d, n_seg)(
        cut_embedding.astype(jnp.float32), idx_pad
    )
    return _normalize(seg_sums.reshape(n_c, n_v, d), lib2d)


# speedup vs baseline: 7.3391x; 1.0355x over previous
import functools

import jax
import jax.numpy as jnp
from jax import lax
from jax.experimental import pallas as pl
from jax.experimental.pallas import tpu as pltpu
from jax.experimental.pallas import tpu_sc as plsc

_SEG_CHUNK = 400
_IDXW = 416
_WIN = 128
_ACC = 408


def _make_sc_segment_sum(n_rows, d, n_seg):
    info = plsc.get_sparse_core_info()
    n_cores, n_sub = info.num_cores, info.num_subcores
    nw = n_cores * n_sub
    nchunk = n_seg // _SEG_CHUNK
    mesh = plsc.VectorSubcoreMesh(core_axis_name="c", subcore_axis_name="s")

    @functools.partial(
        pl.kernel,
        mesh=mesh,
        compiler_params=pltpu.CompilerParams(needs_layout_passes=False),
        out_type=jax.ShapeDtypeStruct((n_seg, d), jnp.float32),
        scratch_types=[
            pltpu.VMEM((_IDXW,), jnp.int32),
            pltpu.VMEM((_WIN, d), jnp.float32),
            pltpu.VMEM((_WIN, d), jnp.float32),
            pltpu.VMEM((_WIN, d), jnp.float32),
            pltpu.VMEM((_WIN,), jnp.int32),
            pltpu.VMEM((_ACC // 3, d), jnp.float32),
            pltpu.VMEM_SHARED((16 * _ACC, d), jnp.float32),
            pltpu.SemaphoreType.DMA,
            pltpu.SemaphoreType.DMA,
            pltpu.SemaphoreType.DMA,
            pltpu.SemaphoreType.DMA,
        ],
    )
    def k(emb_hbm, idx_hbm, out_hbm, idx_v, rows_a, rows_b, rows_c, ids_v,
          zeros_v, acc, sem_a, sem_b, sem_c, sem_o):
        wid = lax.axis_index("s") * n_cores + lax.axis_index("c")
        sid = lax.axis_index("s")
        slot0 = sid * _ACC
        rem = nchunk % nw
        nloc = jnp.where(wid < rem, nchunk // nw + 1, nchunk // nw)

        lane = lax.iota(jnp.int32, 16)
        zrow = jnp.zeros((16,), jnp.float32)

        def lane0_scalar(vec16):
            return vec16[0]

        def zbody(r, c0):
            for g in range(d // 16):
                zeros_v[r, pl.ds(g * 16, 16)] = zrow
            return c0

        lax.fori_loop(0, _ACC // 3, zbody, 0)

        def compute_ids(w):
            wbase = w * _WIN
            for g in range(_WIN // 16):
                rowvec = wbase + g * 16 + lane
                c = jnp.zeros((16,), jnp.int32)
                step = 512
                while step >= 1:
                    probe = c + (step - 1)
                    probe_c = jnp.minimum(probe, _IDXW - 1)
                    vals = plsc.load_gather(idx_v, [probe_c])
                    ok = (vals <= rowvec) & (probe <= _IDXW - 1)
                    c = jnp.where(ok, probe + 1, c)
                    step //= 2
                f = c - 1
                valid = (f >= 0) & (f < _SEG_CHUNK)
                ids_v[pl.ds(g * 16, 16)] = jnp.where(
                    valid, f, _SEG_CHUNK + 4
                ) + slot0

        def issue(w, buf, sem):
            pltpu.async_copy(emb_hbm.at[pl.ds(w * _WIN, _WIN)], buf, sem)

        def drain(w, buf, sem):
            pltpu.make_async_copy(
                emb_hbm.at[pl.ds(w * _WIN, _WIN)], buf, sem
            ).wait()
            pltpu.sync_copy(buf, acc.at[ids_v], add=True)

        def chunk_body(j, carry):
            chunk = wid + j * nw
            base = chunk * _SEG_CHUNK
            pltpu.sync_copy(idx_hbm.at[pl.ds(base, _IDXW)], idx_v)
            r0 = lane0_scalar(idx_v[pl.ds(0, 16)])
            r1 = lane0_scalar(idx_v[pl.ds(_SEG_CHUNK, 16)])
            wlo = r0 // _WIN
            whi = (r1 + (_WIN - 1)) // _WIN
            nwin = whi - wlo

            @pl.when(nwin >= 1)
            def _():
                issue(wlo, rows_a, sem_a)

            @pl.when(nwin >= 2)
            def _():
                issue(wlo + 1, rows_b, sem_b)

            @pl.when(j > 0)
            def _():
                pltpu.make_async_copy(
                    acc.at[pl.ds(slot0, _SEG_CHUNK)],
                    out_hbm.at[pl.ds((chunk - nw) * _SEG_CHUNK, _SEG_CHUNK)],
                    sem_o,
                ).wait()

            for z in range(3):
                pltpu.sync_copy(
                    zeros_v, acc.at[pl.ds(slot0 + z * (_ACC // 3), _ACC // 3)]
                )

            def tri_body(p, c1):
                w = wlo + 3 * p

                @pl.when(w + 2 < whi)
                def _():
                    issue(w + 2, rows_c, sem_c)

                compute_ids(w)
                drain(w, rows_a, sem_a)

                @pl.when(w + 3 < whi)
                def _():
                    issue(w + 3, rows_a, sem_a)

                @pl.when(w + 1 < whi)
                def _():
                    compute_ids(w + 1)
                    drain(w + 1, rows_b, sem_b)

                @pl.when(w + 4 < whi)
                def _():
                    issue(w + 4, rows_b, sem_b)

                @pl.when(w + 2 < whi)
                def _():
                    compute_ids(w + 2)
                    drain(w + 2, rows_c, sem_c)

                return c1

            lax.fori_loop(0, (nwin + 2) // 3, tri_body, 0)
            pltpu.async_copy(
                acc.at[pl.ds(slot0, _SEG_CHUNK)],
                out_hbm.at[pl.ds(base, _SEG_CHUNK)],
                sem_o,
            )
            return carry

        lax.fori_loop(0, nloc, chunk_body, 0)

        @pl.when(nloc > 0)
        def _():
            pltpu.make_async_copy(
                acc.at[pl.ds(slot0, _SEG_CHUNK)],
                out_hbm.at[
                    pl.ds((wid + (nloc - 1) * nw) * _SEG_CHUNK, _SEG_CHUNK)
                ],
                sem_o,
            ).wait()

    return k


_VT = 1000


def _norm_body(s_ref, lib_ref, out_ref):
    s = s_ref[...]
    ve = jnp.log1p(s / lib_ref[...][:, None, :]) - 2.0
    n_c = ve.shape[0]
    mu = jnp.mean(ve, axis=0, keepdims=True)
    sd = jnp.sqrt(jnp.sum((ve - mu) ** 2, axis=0, keepdims=True) / (n_c - 1))
    rel = (ve - mu) / (sd + 1e-5)
    d = ve.shape[-1]
    out_ref[..., 0:d] = ve
    out_ref[..., d : 2 * d] = rel


def _normalize(seg_sums3, lib2d):
    n_c, n_v, d = seg_sums3.shape
    nt = n_v // _VT
    return pl.pallas_call(
        _norm_body,
        grid=(nt,),
        in_specs=[
            pl.BlockSpec((n_c, _VT, d), lambda t: (0, t, 0)),
            pl.BlockSpec((n_c, d), lambda t: (0, 0)),
        ],
        out_specs=pl.BlockSpec((n_c, _VT, 2 * d), lambda t: (0, t, 0)),
        out_shape=jax.ShapeDtypeStruct((n_c, n_v, 2 * d), jnp.float32),
        compiler_params=pltpu.CompilerParams(dimension_semantics=("parallel",)),
    )(seg_sums3, lib2d)


def kernel(cut_embedding, local_clusterxvariant_indptr, n_variants, n_clusters, cluster_cut_lib):
    n_rows, d = cut_embedding.shape
    n_c = cluster_cut_lib.shape[0]
    n_seg = local_clusterxvariant_indptr.shape[0] - 1
    n_v = n_seg // n_c

    idx = local_clusterxvariant_indptr.astype(jnp.int32)
    npad = (n_seg - _SEG_CHUNK) + _IDXW - idx.shape[0]
    idx_pad = jnp.concatenate(
        [idx, jnp.broadcast_to(idx[-1:], (npad,))]
    )
    lib2d = jnp.broadcast_to(
        cluster_cut_lib.astype(jnp.float32)[:, None], (n_c, d)
    )

    seg_sums = _make_sc_segment_sum(n_rows, d, n_seg)(
        cut_embedding.astype(jnp.float32), idx_pad
    )
    return _normalize(seg_sums.reshape(n_c, n_v, d), lib2d)
